# baseline (speedup 1.0000x reference)
import functools

import numpy as np
import jax
import jax.numpy as jnp
from jax import lax
from jax.experimental import pallas as pl
from jax.experimental.pallas import tpu as pltpu
from jax.experimental.pallas import tpu_sc as plsc

_DIM = 3
_START = 384
_END = 704
_NORM = 1000.0
_MAX_CI = 3.0
_BIG = 3.0e38

_NC = 2
_NS = 16


def _cid_body(pa_ref, pb_ref, pc_ref, s_ref, sband_ref, u64_ref, u63_ref,
              idx_ref, p_out_ref):
    p = jnp.concatenate(
        [pa_ref[:, :], pb_ref[:, :], pc_ref[:, :64]], axis=1)
    p_out_ref[:, :] = p
    s = s_ref[0, :]
    nwp = sband_ref.shape[1]
    nw = p.shape[1] - s.shape[0] + 1

    psq = p * p
    dcol = p[:, 1:] - p[:, :-1]
    dsq = dcol * dcol
    dpad = jnp.concatenate(
        [dsq, jnp.zeros((p.shape[0], 1), jnp.float32)], axis=1)

    hi = lax.Precision.HIGHEST
    q = jnp.dot(psq, u64_ref[:, :], precision=hi)
    dw = jnp.dot(dpad, u63_ref[:, :], precision=hi)
    c = jnp.dot(p, sband_ref[:, :], precision=hi)

    ssum = jnp.sum(s * s)
    sd = s[1:] - s[:-1]
    sci = jnp.sqrt(jnp.sum(sd * sd) + 1.0 / _NORM)

    ed = jnp.sqrt(jnp.maximum(q - 2.0 * c + ssum, 0.0))
    pci = jnp.sqrt(dw + 1.0 / _NORM)
    cf = jnp.minimum(jnp.maximum(pci, sci) / jnp.minimum(pci, sci), _MAX_CI)
    cid = ed * cf
    col = lax.broadcasted_iota(jnp.int32, cid.shape, 1)
    cid = jnp.where(col < nw, cid, _BIG)
    m = jnp.min(cid, axis=1, keepdims=True)
    cand = jnp.where(cid == m, col, jnp.int32(2**30))
    idx_ref[:, :] = jnp.min(cand, axis=1, keepdims=True)


def _make_retrieve(batch, pl_len, ls):
    nworkers = _NC * _NS
    rpw = batch // nworkers

    mesh = plsc.VectorSubcoreMesh(
        core_axis_name="c", subcore_axis_name="s",
        num_cores=_NC, num_subcores=_NS)

    @functools.partial(
        pl.kernel, mesh=mesh,
        compiler_params=pltpu.CompilerParams(
            needs_layout_passes=False, skip_device_barrier=True),
        out_type=jax.ShapeDtypeStruct((batch * ls,), jnp.float32),
        scratch_types=[
            pltpu.VMEM((rpw * pl_len,), jnp.float32),
            pltpu.VMEM((rpw + 16,), jnp.int32),
            pltpu.VMEM((rpw * ls,), jnp.float32),
        ],
    )
    def retrieve(p_hbm, idx_hbm, out_hbm, pv, iv, wv):
        wid = lax.axis_index("s") * _NC + lax.axis_index("c")
        base = wid * rpw
        pltpu.sync_copy(p_hbm.at[pl.ds(base * pl_len, rpw * pl_len)], pv)
        pltpu.sync_copy(idx_hbm.at[pl.ds(base, rpw)], iv.at[pl.ds(0, rpw)])

        lane = lax.iota(jnp.int32, 16)
        zero16 = jnp.zeros((16,), jnp.int32)
        dnums = lax.GatherDimensionNumbers(
            offset_dims=(), collapsed_slice_dims=(0,), start_index_map=(0,))

        def row(r, carry):
            ivec = iv[pl.ds(r, 16)]
            bc = lax.gather(ivec, zero16[:, None], dnums, (1,),
                            mode=lax.GatherScatterMode.PROMISE_IN_BOUNDS)
            rbase = bc + r * pl_len
            for j in range(ls // 16):
                inds = lane + rbase + (j * 16)
                wv[pl.ds(r * ls + j * 16, 16)] = plsc.load_gather(pv, [inds])
            return carry

        lax.fori_loop(0, rpw, row, 0)
        pltpu.sync_copy(wv, out_hbm.at[pl.ds(base * ls, rpw * ls)])

    return retrieve


def _embed_body(w_ref, W1_ref, b1_ref, s_ref, W2_ref, b2_ref, o_ref):
    hi = lax.Precision.HIGHEST
    win = w_ref[:, :]
    out_s = jnp.dot(s_ref[:, :], W2_ref[:, :].T, precision=hi) + b2_ref[0, :]
    out_i = jnp.dot(win, W1_ref[:, :].T, precision=hi) + b1_ref[0, :]
    o_ref[:, :] = out_i - out_s[0, :]


def kernel(x, shapelet, W1, b1, W2, b2):
    batch = x.shape[0]
    pl_len = _END - _START
    ls = shapelet.shape[0]
    nw = pl_len - ls + 1
    nwp = 384
    emb = W1.shape[0]
    bb = 256

    xf = x.reshape(batch, x.shape[1] * x.shape[2])
    off = _DIM * x.shape[2] + _START

    ti = np.arange(pl_len)[:, None]
    wi = np.arange(nwp)[None, :]
    rel = ti - wi
    u64 = jnp.asarray(
        ((rel >= 0) & (rel < ls) & (wi < nw)).astype(np.float32))
    u63 = jnp.asarray(
        ((rel >= 0) & (rel < ls - 1) & (wi < nw)).astype(np.float32))
    per = pl_len + nwp
    fv = jnp.zeros((per,), jnp.float32)
    fv = lax.dynamic_update_slice(fv, shapelet, (nwp - 1,))
    w2 = jnp.tile(fv, pl_len + 1)[: pl_len * (per + 1)].reshape(
        pl_len, per + 1)
    sband = w2[:, :nwp][:, ::-1]
    s2d = shapelet.reshape(1, ls)

    idx, piss = pl.pallas_call(
        _cid_body,
        grid=(batch // bb,),
        in_specs=[
            pl.BlockSpec((bb, 128), lambda i: (i, off // 128)),
            pl.BlockSpec((bb, 128), lambda i: (i, off // 128 + 1)),
            pl.BlockSpec((bb, 128), lambda i: (i, off // 128 + 2)),
            pl.BlockSpec((1, ls), lambda i: (0, 0)),
            pl.BlockSpec((pl_len, nwp), lambda i: (0, 0)),
            pl.BlockSpec((pl_len, nwp), lambda i: (0, 0)),
            pl.BlockSpec((pl_len, nwp), lambda i: (0, 0)),
        ],
        out_specs=[
            pl.BlockSpec((bb, 1), lambda i: (i, 0)),
            pl.BlockSpec((bb, pl_len), lambda i: (i, 0)),
        ],
        out_shape=[
            jax.ShapeDtypeStruct((batch, 1), jnp.int32),
            jax.ShapeDtypeStruct((batch, pl_len), jnp.float32),
        ],
    )(xf, xf, xf, s2d, sband, u64, u63)

    retrieve = _make_retrieve(batch, pl_len, ls)
    win = retrieve(piss.reshape(-1), idx.reshape(-1)).reshape(batch, ls)

    out = pl.pallas_call(
        _embed_body,
        grid=(batch // bb,),
        in_specs=[
            pl.BlockSpec((bb, ls), lambda i: (i, 0)),
            pl.BlockSpec((emb, ls), lambda i: (0, 0)),
            pl.BlockSpec((1, emb), lambda i: (0, 0)),
            pl.BlockSpec((1, ls), lambda i: (0, 0)),
            pl.BlockSpec((emb, ls), lambda i: (0, 0)),
            pl.BlockSpec((1, emb), lambda i: (0, 0)),
        ],
        out_specs=pl.BlockSpec((bb, emb), lambda i: (i, 0)),
        out_shape=jax.ShapeDtypeStruct((batch, emb), jnp.float32),
    )(win, W1, b1.reshape(1, emb), s2d, W2, b2.reshape(1, emb))

    return out.reshape(batch, 1, emb)


# speedup vs baseline: 1.2675x; 1.2675x over previous
import functools

import numpy as np
import jax
import jax.numpy as jnp
from jax import lax
from jax.experimental import pallas as pl
from jax.experimental.pallas import tpu as pltpu
from jax.experimental.pallas import tpu_sc as plsc

_DIM = 3
_START = 384
_END = 704
_NORM = 1000.0
_MAX_CI = 3.0
_BIG = 3.0e38

_NC = 2
_NS = 16


def _cid_body(p_ref, s_ref, sband_ref, u64_ref, u63_ref, idx_ref):
    p = p_ref[:, :]
    s = s_ref[0, :]
    nwp = sband_ref.shape[1]
    nw = p.shape[1] - s.shape[0] + 1

    psq = p * p
    dcol = p[:, 1:] - p[:, :-1]
    dsq = dcol * dcol
    dpad = jnp.concatenate(
        [dsq, jnp.zeros((p.shape[0], 1), jnp.float32)], axis=1)

    hi = lax.Precision.HIGHEST
    q = jnp.dot(psq, u64_ref[:, :], precision=hi)
    dw = jnp.dot(dpad, u63_ref[:, :], precision=hi)
    c = jnp.dot(p, sband_ref[:, :], precision=hi)

    ssum = jnp.sum(s * s)
    sd = s[1:] - s[:-1]
    sci = jnp.sqrt(jnp.sum(sd * sd) + 1.0 / _NORM)

    ed = jnp.sqrt(jnp.maximum(q - 2.0 * c + ssum, 0.0))
    pci = jnp.sqrt(dw + 1.0 / _NORM)
    cf = jnp.minimum(jnp.maximum(pci, sci) / jnp.minimum(pci, sci), _MAX_CI)
    cid = ed * cf
    col = lax.broadcasted_iota(jnp.int32, cid.shape, 1)
    cid = jnp.where(col < nw, cid, _BIG)
    m = jnp.min(cid, axis=1, keepdims=True)
    cand = jnp.where(cid == m, col, jnp.int32(2**30))
    idx_ref[:, :] = jnp.min(cand, axis=1, keepdims=True)


def _make_retrieve(batch, pl_len, ls):
    nworkers = _NC * _NS
    rpw = batch // nworkers

    mesh = plsc.VectorSubcoreMesh(
        core_axis_name="c", subcore_axis_name="s",
        num_cores=_NC, num_subcores=_NS)

    @functools.partial(
        pl.kernel, mesh=mesh,
        compiler_params=pltpu.CompilerParams(
            needs_layout_passes=False, skip_device_barrier=True),
        out_type=jax.ShapeDtypeStruct((batch * ls,), jnp.float32),
        scratch_types=[
            pltpu.VMEM((rpw * pl_len,), jnp.float32),
            pltpu.VMEM((rpw + 16,), jnp.int32),
            pltpu.VMEM((rpw * ls,), jnp.float32),
        ],
    )
    def retrieve(p_hbm, idx_hbm, out_hbm, pv, iv, wv):
        wid = lax.axis_index("s") * _NC + lax.axis_index("c")
        base = wid * rpw
        pltpu.sync_copy(p_hbm.at[pl.ds(base * pl_len, rpw * pl_len)], pv)
        pltpu.sync_copy(idx_hbm.at[pl.ds(base, rpw)], iv.at[pl.ds(0, rpw)])

        lane = lax.iota(jnp.int32, 16)
        zero16 = jnp.zeros((16,), jnp.int32)
        dnums = lax.GatherDimensionNumbers(
            offset_dims=(), collapsed_slice_dims=(0,), start_index_map=(0,))

        def row(r, carry):
            ivec = iv[pl.ds(r, 16)]
            bc = lax.gather(ivec, zero16[:, None], dnums, (1,),
                            mode=lax.GatherScatterMode.PROMISE_IN_BOUNDS)
            rbase = bc + r * pl_len
            for j in range(ls // 16):
                inds = lane + rbase + (j * 16)
                wv[pl.ds(r * ls + j * 16, 16)] = plsc.load_gather(pv, [inds])
            return carry

        lax.fori_loop(0, rpw, row, 0)
        pltpu.sync_copy(wv, out_hbm.at[pl.ds(base * ls, rpw * ls)])

    return retrieve


def _embed_body(w_ref, W1_ref, b1_ref, s_ref, W2_ref, b2_ref, o_ref):
    hi = lax.Precision.HIGHEST
    win = w_ref[:, :]
    out_s = jnp.dot(s_ref[:, :], W2_ref[:, :].T, precision=hi) + b2_ref[0, :]
    out_i = jnp.dot(win, W1_ref[:, :].T, precision=hi) + b1_ref[0, :]
    o_ref[:, :] = out_i - out_s[0, :]


def kernel(x, shapelet, W1, b1, W2, b2):
    batch = x.shape[0]
    pl_len = _END - _START
    ls = shapelet.shape[0]
    nw = pl_len - ls + 1
    nwp = 384
    emb = W1.shape[0]
    bb = 256

    piss = x[:, _DIM, _START:_END]

    ti = np.arange(pl_len)[:, None]
    wi = np.arange(nwp)[None, :]
    rel = ti - wi
    u64 = jnp.asarray(
        ((rel >= 0) & (rel < ls) & (wi < nw)).astype(np.float32))
    u63 = jnp.asarray(
        ((rel >= 0) & (rel < ls - 1) & (wi < nw)).astype(np.float32))
    per = pl_len + nwp
    fv = jnp.zeros((per,), jnp.float32)
    fv = lax.dynamic_update_slice(fv, shapelet, (nwp - 1,))
    w2 = jnp.tile(fv, pl_len + 1)[: pl_len * (per + 1)].reshape(
        pl_len, per + 1)
    sband = w2[:, :nwp][:, ::-1]
    s2d = shapelet.reshape(1, ls)

    idx = pl.pallas_call(
        _cid_body,
        grid=(batch // bb,),
        in_specs=[
            pl.BlockSpec((bb, pl_len), lambda i: (i, 0)),
            pl.BlockSpec((1, ls), lambda i: (0, 0)),
            pl.BlockSpec((pl_len, nwp), lambda i: (0, 0)),
            pl.BlockSpec((pl_len, nwp), lambda i: (0, 0)),
            pl.BlockSpec((pl_len, nwp), lambda i: (0, 0)),
        ],
        out_specs=pl.BlockSpec((bb, 1), lambda i: (i, 0)),
        out_shape=jax.ShapeDtypeStruct((batch, 1), jnp.int32),
    )(piss, s2d, sband, u64, u63)

    half = batch // 2
    retrieve = _make_retrieve(half, pl_len, ls)
    pf = piss.reshape(-1)
    ixf = idx.reshape(-1)
    win0 = retrieve(pf[: half * pl_len], ixf[:half])
    win1 = retrieve(pf[half * pl_len:], ixf[half:])
    win = jnp.concatenate([win0, win1]).reshape(batch, ls)

    out = pl.pallas_call(
        _embed_body,
        grid=(batch // bb,),
        in_specs=[
            pl.BlockSpec((bb, ls), lambda i: (i, 0)),
            pl.BlockSpec((emb, ls), lambda i: (0, 0)),
            pl.BlockSpec((1, emb), lambda i: (0, 0)),
            pl.BlockSpec((1, ls), lambda i: (0, 0)),
            pl.BlockSpec((emb, ls), lambda i: (0, 0)),
            pl.BlockSpec((1, emb), lambda i: (0, 0)),
        ],
        out_specs=pl.BlockSpec((bb, emb), lambda i: (i, 0)),
        out_shape=jax.ShapeDtypeStruct((batch, emb), jnp.float32),
    )(win, W1, b1.reshape(1, emb), s2d, W2, b2.reshape(1, emb))

    return out.reshape(batch, 1, emb)


# speedup vs baseline: 1.5238x; 1.2022x over previous
import functools

import numpy as np
import jax
import jax.numpy as jnp
from jax import lax
from jax.experimental import pallas as pl
from jax.experimental.pallas import tpu as pltpu
from jax.experimental.pallas import tpu_sc as plsc

_DIM = 3
_START = 384
_END = 704
_NORM = 1000.0
_MAX_CI = 3.0
_BIG = 3.0e38

_NC = 2
_NS = 16


def _cid_body(p_ref, s_ref, sband_ref, u64_ref, u63_ref, idx_ref):
    p = p_ref[:, :]
    s = s_ref[0, :]
    nwp = sband_ref.shape[1]
    nw = p.shape[1] - s.shape[0] + 1

    bbsz = p.shape[0]
    psq = p * p
    dcol = p[:, 1:] - p[:, :-1]
    dsq = dcol * dcol
    dpad = jnp.concatenate(
        [dsq, jnp.zeros((bbsz, 1), jnp.float32)], axis=1)

    zpad = jnp.zeros((bbsz, 64), jnp.bfloat16)

    def split3(v):
        a1 = v.astype(jnp.bfloat16)
        r1 = v - a1.astype(jnp.float32)
        a2 = r1.astype(jnp.bfloat16)
        r2 = r1 - a2.astype(jnp.float32)
        a3 = r2.astype(jnp.bfloat16)
        return jnp.concatenate([a1, zpad, a2, zpad, a3, zpad], axis=1)

    f32 = jnp.float32
    q = jnp.dot(split3(psq), u64_ref[:, :], preferred_element_type=f32)
    dw = jnp.dot(split3(dpad), u63_ref[:, :], preferred_element_type=f32)
    c = jnp.dot(p, sband_ref[:, :], precision=lax.Precision.HIGHEST)

    ssum = jnp.sum(s * s)
    sd = s[1:] - s[:-1]
    sci = jnp.sqrt(jnp.sum(sd * sd) + 1.0 / _NORM)

    ed = jnp.sqrt(jnp.maximum(q - 2.0 * c + ssum, 0.0))
    pci = jnp.sqrt(dw + 1.0 / _NORM)
    cf = jnp.minimum(jnp.maximum(pci, sci) / jnp.minimum(pci, sci), _MAX_CI)
    cid = ed * cf
    col = lax.broadcasted_iota(jnp.int32, cid.shape, 1)
    cid = jnp.where(col < nw, cid, _BIG)
    m = jnp.min(cid, axis=1, keepdims=True)
    cand = jnp.where(cid == m, col, jnp.int32(2**30))
    idx_ref[:, :] = jnp.min(cand, axis=1, keepdims=True)


def _make_retrieve(batch, pl_len, ls):
    nworkers = _NC * _NS
    rpw = batch // nworkers

    mesh = plsc.VectorSubcoreMesh(
        core_axis_name="c", subcore_axis_name="s",
        num_cores=_NC, num_subcores=_NS)

    @functools.partial(
        pl.kernel, mesh=mesh,
        compiler_params=pltpu.CompilerParams(
            needs_layout_passes=False, skip_device_barrier=True),
        out_type=jax.ShapeDtypeStruct((batch * ls,), jnp.float32),
        scratch_types=[
            pltpu.VMEM((rpw * pl_len,), jnp.float32),
            pltpu.VMEM((rpw + 16,), jnp.int32),
            pltpu.VMEM((rpw * ls,), jnp.float32),
            pltpu.SemaphoreType.DMA,
            pltpu.SemaphoreType.DMA,
        ],
    )
    def retrieve(p_hbm, idx_hbm, out_hbm, pv, iv, wv, sem_p, sem_i):
        wid = lax.axis_index("s") * _NC + lax.axis_index("c")
        base = wid * rpw
        cp = pltpu.async_copy(
            p_hbm.at[pl.ds(base * pl_len, rpw * pl_len)], pv, sem_p)
        ci = pltpu.async_copy(
            idx_hbm.at[pl.ds(base, rpw)], iv.at[pl.ds(0, rpw)], sem_i)
        ci.wait()
        cp.wait()

        lane = lax.iota(jnp.int32, 16)
        zero16 = jnp.zeros((16,), jnp.int32)
        dnums = lax.GatherDimensionNumbers(
            offset_dims=(), collapsed_slice_dims=(0,), start_index_map=(0,))

        def row(r, carry):
            ivec = iv[pl.ds(r, 16)]
            bc = lax.gather(ivec, zero16[:, None], dnums, (1,),
                            mode=lax.GatherScatterMode.PROMISE_IN_BOUNDS)
            rbase = bc + r * pl_len
            for j in range(ls // 16):
                inds = lane + rbase + (j * 16)
                wv[pl.ds(r * ls + j * 16, 16)] = plsc.load_gather(pv, [inds])
            return carry

        lax.fori_loop(0, rpw, row, 0)
        pltpu.sync_copy(wv, out_hbm.at[pl.ds(base * ls, rpw * ls)])

    return retrieve


def _embed_body(w_ref, W1_ref, b1_ref, s_ref, W2_ref, b2_ref, o_ref):
    hi = lax.Precision.HIGHEST
    win = w_ref[:, :]
    out_s = jnp.dot(s_ref[:, :], W2_ref[:, :].T, precision=hi) + b2_ref[0, :]
    out_i = jnp.dot(win, W1_ref[:, :].T, precision=hi) + b1_ref[0, :]
    o_ref[:, :] = out_i - out_s[0, :]


def kernel(x, shapelet, W1, b1, W2, b2):
    batch = x.shape[0]
    pl_len = _END - _START
    ls = shapelet.shape[0]
    nw = pl_len - ls + 1
    nwp = 384
    emb = W1.shape[0]
    bb = 256

    piss = x[:, _DIM, _START:_END]

    ti = np.arange(pl_len)[:, None]
    wi = np.arange(nwp)[None, :]
    rel = ti - wi
    u64np = ((rel >= 0) & (rel < ls) & (wi < nw)).astype(np.float32)
    u63np = ((rel >= 0) & (rel < ls - 1) & (wi < nw)).astype(np.float32)
    zrow = np.zeros((64, nwp), np.float32)

    def stack3(m):
        st = np.concatenate([m, zrow, m, zrow, m, zrow], axis=0)
        return jnp.asarray(st).astype(jnp.bfloat16)

    u64 = stack3(u64np)
    u63 = stack3(u63np)
    per = pl_len + nwp
    fv = jnp.zeros((per,), jnp.float32)
    fv = lax.dynamic_update_slice(fv, shapelet, (nwp - 1,))
    w2 = jnp.tile(fv, pl_len + 1)[: pl_len * (per + 1)].reshape(
        pl_len, per + 1)
    sband = w2[:, :nwp][:, ::-1]
    s2d = shapelet.reshape(1, ls)

    idx = pl.pallas_call(
        _cid_body,
        grid=(batch // bb,),
        in_specs=[
            pl.BlockSpec((bb, pl_len), lambda i: (i, 0)),
            pl.BlockSpec((1, ls), lambda i: (0, 0)),
            pl.BlockSpec((pl_len, nwp), lambda i: (0, 0)),
            pl.BlockSpec((3 * (pl_len + 64), nwp), lambda i: (0, 0)),
            pl.BlockSpec((3 * (pl_len + 64), nwp), lambda i: (0, 0)),
        ],
        out_specs=pl.BlockSpec((bb, 1), lambda i: (i, 0)),
        out_shape=jax.ShapeDtypeStruct((batch, 1), jnp.int32),
    )(piss, s2d, sband, u64, u63)

    retrieve = _make_retrieve(batch, pl_len, ls)
    win = retrieve(piss.reshape(-1), idx.reshape(-1)).reshape(batch, ls)

    out = pl.pallas_call(
        _embed_body,
        grid=(batch // bb,),
        in_specs=[
            pl.BlockSpec((bb, ls), lambda i: (i, 0)),
            pl.BlockSpec((emb, ls), lambda i: (0, 0)),
            pl.BlockSpec((1, emb), lambda i: (0, 0)),
            pl.BlockSpec((1, ls), lambda i: (0, 0)),
            pl.BlockSpec((emb, ls), lambda i: (0, 0)),
            pl.BlockSpec((1, emb), lambda i: (0, 0)),
        ],
        out_specs=pl.BlockSpec((bb, emb), lambda i: (i, 0)),
        out_shape=jax.ShapeDtypeStruct((batch, emb), jnp.float32),
    )(win, W1, b1.reshape(1, emb), s2d, W2, b2.reshape(1, emb))

    return out.reshape(batch, 1, emb)
